# baseline (speedup 1.0000x reference)
import jax
import jax.numpy as jnp
from jax import lax
from jax.experimental import pallas as pl
from jax.experimental.pallas import tpu as pltpu
from jax.experimental.pallas import tpu_sc as plsc

BATCH = 16
HEIGHT = 90
LENGTH = 16384
NC = 2
NS = 16
LANES = 16
BPC = BATCH // NC

ROWS_PER_W = HEIGHT // 2
NBUF = 3
UNROLL = 8
NACC = 4
SUBL = LENGTH // (UNROLL * LANES)

assert ROWS_PER_W % NBUF == 0

C1 = 6.020599913279624
C2 = 3.010299956639812
INV_LN2 = 1.4426950408889634


def _accumulate_chunk(cs_b, csp_b, mask_v, accs, cnts):
    def inner(i, carry):
        acc, cnt = carry
        acc = list(acc)
        cnt = list(cnt)
        for u in range(UNROLL):
            j = u % NACC
            a = cs_b[pl.ds(i * (UNROLL * LANES) + u * LANES, LANES)]
            p = csp_b[i, pl.ds(u * LANES, LANES)]
            mf = mask_v[i, pl.ds(u * LANES, LANES)]
            notnan = a == a
            asafe = jnp.where(notnan, a, p)
            d = asafe - p
            dm = d * mf
            acc[j] = acc[j] + dm * d
            cnt[j] = cnt[j] + jnp.where(notnan, mf, 0.0)
        return (tuple(acc), tuple(cnt))

    return lax.fori_loop(0, SUBL, inner, (accs, cnts))


def _log2(x):
    bits = plsc.bitcast(x, jnp.int32)
    e = ((bits >> 23) & 0xFF) - 127
    m = plsc.bitcast((bits & 0x007FFFFF) | 0x3F800000, jnp.float32)
    s = (m - 1.0) / (m + 1.0)
    s2 = s * s
    ln_m = 2.0 * s * (1.0 + s2 * (1.0 / 3.0 + s2 * (0.2 + s2 * (1.0 / 7.0))))
    return e.astype(jnp.float32) + ln_m * INV_LN2


def _sc_body(cs_hbm, csp_hbm, mask_hbm, psnr_out,
             mask_v, cs0, cs1, cs2, csp0, csp1, csp2, stage_v, comb_v,
             shared, sem0, sem1, sem2):
    c = lax.axis_index("c")
    s = lax.axis_index("s")
    batch = c * BPC + (s // 2)
    half = s % 2
    row0 = half * ROWS_PER_W

    pltpu.sync_copy(mask_hbm.at[batch], mask_v)

    def mask_body16(i, _):
        for u in range(UNROLL):
            sl = pl.ds(u * LANES, LANES)
            m = mask_v[i, sl]
            mask_v[i, sl] = jnp.where(m > 0.0, 1.0, 0.0)
        return 0

    lax.fori_loop(0, SUBL, mask_body16, 0)

    bufs = ((cs0, csp0, sem0), (cs1, csp1, sem1), (cs2, csp2, sem2))

    def start(j, b):
        cs_b, csp_b, sem = bufs[b]
        pltpu.async_copy(cs_hbm.at[batch, row0 + j], cs_b, sem)
        pltpu.async_copy(csp_hbm.at[batch, row0 + j], csp_b, sem)

    def wait(b):
        cs_b, csp_b, sem = bufs[b]
        pltpu.make_async_copy(cs_hbm.at[batch, row0], cs_b, sem).wait()
        pltpu.make_async_copy(csp_hbm.at[batch, row0], csp_b, sem).wait()

    for b in range(NBUF):
        start(b, b)

    def body3(k, carry):
        accs, cnts = carry
        for b in range(NBUF):
            wait(b)
            accs, cnts = _accumulate_chunk(bufs[b][0], bufs[b][1], mask_v,
                                           accs, cnts)
            start(NBUF * k + b + NBUF, b)
        return (accs, cnts)

    zero = jnp.zeros((LANES,), jnp.float32)
    init = (tuple(zero for _ in range(NACC)), tuple(zero for _ in range(NACC)))
    accs, cnts = lax.fori_loop(0, ROWS_PER_W // NBUF - 1, body3, init)

    for b in range(NBUF):
        wait(b)
        accs, cnts = _accumulate_chunk(bufs[b][0], bufs[b][1], mask_v,
                                       accs, cnts)

    stage_v[...] = (accs[0] + accs[1]) + (accs[2] + accs[3])
    pltpu.sync_copy(stage_v, shared.at[s, 0])
    stage_v[...] = (cnts[0] + cnts[1]) + (cnts[2] + cnts[3])
    pltpu.sync_copy(stage_v, shared.at[s, 1])
    plsc.subcore_barrier()

    @pl.when(s == 0)
    def _():
        pltpu.sync_copy(shared, comb_v)
        laneid = lax.iota(jnp.int32, LANES)
        vssq = jnp.zeros((LANES,), jnp.float32)
        vcnt = jnp.zeros((LANES,), jnp.float32)
        for k in range(BPC):
            ssq_k = jnp.sum(comb_v[2 * k, 0] + comb_v[2 * k + 1, 0])
            cnt_k = jnp.sum(comb_v[2 * k, 1] + comb_v[2 * k + 1, 1])
            vssq = jnp.where(laneid == k, ssq_k, vssq)
            vcnt = jnp.where(laneid == k, cnt_k, vcnt)
        mse = vssq / vcnt
        psnr = C1 - C2 * _log2(mse)
        psnr = jnp.where(mse == 0.0, jnp.float32(jnp.inf), psnr)
        psnr = jnp.where(laneid < BPC, psnr, 0.0)
        core_sum = jnp.sum(psnr)
        stage_v[...] = jnp.where(laneid == 0, core_sum, 0.0)
        pltpu.sync_copy(stage_v, psnr_out.at[c])


def _sc_psnr(cs3d, csp4d, mask3d):
    mesh = plsc.VectorSubcoreMesh(core_axis_name="c", subcore_axis_name="s")
    kern = pl.kernel(
        _sc_body,
        out_type=jax.ShapeDtypeStruct((NC, LANES), jnp.float32),
        mesh=mesh,
        scratch_types=[
            pltpu.VMEM((SUBL, UNROLL * LANES), jnp.float32),
            pltpu.VMEM((LENGTH,), jnp.float32),
            pltpu.VMEM((LENGTH,), jnp.float32),
            pltpu.VMEM((LENGTH,), jnp.float32),
            pltpu.VMEM((SUBL, UNROLL * LANES), jnp.float32),
            pltpu.VMEM((SUBL, UNROLL * LANES), jnp.float32),
            pltpu.VMEM((SUBL, UNROLL * LANES), jnp.float32),
            pltpu.VMEM((LANES,), jnp.float32),
            pltpu.VMEM((NS, 2, LANES), jnp.float32),
            pltpu.VMEM_SHARED((NS, 2, LANES), jnp.float32),
            pltpu.SemaphoreType.DMA,
            pltpu.SemaphoreType.DMA,
            pltpu.SemaphoreType.DMA,
        ],
        compiler_params=pltpu.CompilerParams(needs_layout_passes=False),
    )
    return kern(cs3d, csp4d, mask3d)


def kernel(cs, cs_p, overpass_mask):
    assert cs.shape == (BATCH, HEIGHT, LENGTH)
    psnr_sums = _sc_psnr(cs, cs_p, overpass_mask)
    return (psnr_sums[0, 0] + psnr_sums[1, 0]) / BATCH


# speedup vs baseline: 1.0104x; 1.0104x over previous
import jax
import jax.numpy as jnp
from jax import lax
from jax.experimental import pallas as pl
from jax.experimental.pallas import tpu as pltpu
from jax.experimental.pallas import tpu_sc as plsc

BATCH = 16
HEIGHT = 90
LENGTH = 16384
NC = 2
NS = 16
LANES = 16

ROWS_PER_W = HEIGHT // NC
NBUF = 3
UNROLL = 8
NACC = 4
SUBL = LENGTH // (UNROLL * LANES)

assert ROWS_PER_W % NBUF == 0


def _accumulate_chunk(cs_b, csp_b, mask_v, accs, cnts):
    def inner(i, carry):
        acc, cnt = carry
        acc = list(acc)
        cnt = list(cnt)
        for u in range(UNROLL):
            j = u % NACC
            a = cs_b[pl.ds(i * (UNROLL * LANES) + u * LANES, LANES)]
            p = csp_b[i, pl.ds(u * LANES, LANES)]
            mf = mask_v[i, pl.ds(u * LANES, LANES)]
            notnan = a == a
            asafe = jnp.where(notnan, a, p)
            d = asafe - p
            dm = d * mf
            acc[j] = acc[j] + dm * d
            cnt[j] = cnt[j] + jnp.where(notnan, mf, 0.0)
        return (tuple(acc), tuple(cnt))

    return lax.fori_loop(0, SUBL, inner, (accs, cnts))


def _sc_body(cs_hbm, csp_hbm, mask_hbm, sumsq_out, cnt_out,
             mask_v, cs0, cs1, cs2, csp0, csp1, csp2, stage_v,
             sem0, sem1, sem2):
    c = lax.axis_index("c")
    s = lax.axis_index("s")
    batch = s
    half = c
    row0 = half * ROWS_PER_W

    pltpu.sync_copy(mask_hbm.at[batch], mask_v)

    def mask_body16(i, _):
        for u in range(UNROLL):
            sl = pl.ds(u * LANES, LANES)
            m = mask_v[i, sl]
            mask_v[i, sl] = jnp.where(m > 0.0, 1.0, 0.0)
        return 0

    lax.fori_loop(0, SUBL, mask_body16, 0)

    bufs = ((cs0, csp0, sem0), (cs1, csp1, sem1), (cs2, csp2, sem2))

    def start(j, b):
        cs_b, csp_b, sem = bufs[b]
        pltpu.async_copy(cs_hbm.at[batch, row0 + j], cs_b, sem)
        pltpu.async_copy(csp_hbm.at[batch, row0 + j], csp_b, sem)

    def wait(b):
        cs_b, csp_b, sem = bufs[b]
        pltpu.make_async_copy(cs_hbm.at[batch, row0], cs_b, sem).wait()
        pltpu.make_async_copy(csp_hbm.at[batch, row0], csp_b, sem).wait()

    for b in range(NBUF):
        start(b, b)

    def body3(k, carry):
        accs, cnts = carry
        for b in range(NBUF):
            wait(b)
            accs, cnts = _accumulate_chunk(bufs[b][0], bufs[b][1], mask_v,
                                           accs, cnts)
            start(NBUF * k + b + NBUF, b)
        return (accs, cnts)

    zero = jnp.zeros((LANES,), jnp.float32)
    init = (tuple(zero for _ in range(NACC)), tuple(zero for _ in range(NACC)))
    accs, cnts = lax.fori_loop(0, ROWS_PER_W // NBUF - 1, body3, init)

    for b in range(NBUF):
        wait(b)
        accs, cnts = _accumulate_chunk(bufs[b][0], bufs[b][1], mask_v,
                                       accs, cnts)

    stage_v[...] = (accs[0] + accs[1]) + (accs[2] + accs[3])
    pltpu.sync_copy(stage_v, sumsq_out.at[half, batch])
    stage_v[...] = (cnts[0] + cnts[1]) + (cnts[2] + cnts[3])
    pltpu.sync_copy(stage_v, cnt_out.at[half, batch])


def _sc_partials(cs3d, csp4d, mask3d):
    mesh = plsc.VectorSubcoreMesh(core_axis_name="c", subcore_axis_name="s")
    kern = pl.kernel(
        _sc_body,
        out_type=(
            jax.ShapeDtypeStruct((NC, BATCH, LANES), jnp.float32),
            jax.ShapeDtypeStruct((NC, BATCH, LANES), jnp.float32),
        ),
        mesh=mesh,
        scratch_types=[
            pltpu.VMEM((SUBL, UNROLL * LANES), jnp.float32),
            pltpu.VMEM((LENGTH,), jnp.float32),
            pltpu.VMEM((LENGTH,), jnp.float32),
            pltpu.VMEM((LENGTH,), jnp.float32),
            pltpu.VMEM((SUBL, UNROLL * LANES), jnp.float32),
            pltpu.VMEM((SUBL, UNROLL * LANES), jnp.float32),
            pltpu.VMEM((SUBL, UNROLL * LANES), jnp.float32),
            pltpu.VMEM((LANES,), jnp.float32),
            pltpu.SemaphoreType.DMA,
            pltpu.SemaphoreType.DMA,
            pltpu.SemaphoreType.DMA,
        ],
    )
    return kern(cs3d, csp4d, mask3d)


def _finalize_body(sumsq_ref, cnt_ref, out_ref):
    ssq = sumsq_ref[0] + sumsq_ref[1]
    cnt = cnt_ref[0] + cnt_ref[1]
    ssq_b = jnp.sum(ssq, axis=1)
    cnt_b = jnp.sum(cnt, axis=1)
    mse = ssq_b / cnt_b
    psnr = jnp.where(mse == 0.0, jnp.inf,
                     20.0 * jnp.log10(2.0 / jnp.sqrt(mse)))
    out_ref[...] = (jnp.sum(psnr) / BATCH).reshape(1, 1)


def _finalize(sumsq, cnt):
    return pl.pallas_call(
        _finalize_body,
        out_shape=jax.ShapeDtypeStruct((1, 1), jnp.float32),
    )(sumsq, cnt)


def kernel(cs, cs_p, overpass_mask):
    assert cs.shape == (BATCH, HEIGHT, LENGTH)
    sumsq, cnt = _sc_partials(cs, cs_p, overpass_mask)
    return _finalize(sumsq, cnt)[0, 0]
